# baseline (speedup 1.0000x reference)
import functools

import jax
import jax.numpy as jnp
from jax import lax
from jax.experimental import pallas as pl
from jax.experimental.pallas import tpu as pltpu
from jax.experimental.pallas import tpu_sc as plsc

E = 8
L = 16
CG = 8
CB = 512
OB = CB * E


def _make_sc_unpool(num: int, channel: int):
    info = plsc.get_sparse_core_info()
    nc, ns = info.num_cores, info.num_subcores
    nw = nc * ns
    ngroups = channel // CG
    P = nw // ngroups
    nb_full = num // CB
    rem = num - nb_full * CB
    kmax = (nb_full + P - 1) // P

    mesh = plsc.VectorSubcoreMesh(core_axis_name="c", subcore_axis_name="s")

    @functools.partial(
        pl.kernel,
        mesh=mesh,
        compiler_params=pltpu.CompilerParams(needs_layout_passes=False),
        out_type=jax.ShapeDtypeStruct((channel, num * E), jnp.float32),
        scratch_types=[
            pltpu.VMEM((CG, CB), jnp.float32),
            pltpu.VMEM((CG, CB), jnp.float32),
            pltpu.VMEM((CG, CB), jnp.int32),
            pltpu.VMEM((CG, CB), jnp.int32),
            pltpu.VMEM((CG, OB), jnp.float32),
            pltpu.VMEM((CG, OB), jnp.float32),
            pltpu.SemaphoreType.DMA,
            pltpu.SemaphoreType.DMA,
            pltpu.SemaphoreType.DMA,
            pltpu.SemaphoreType.DMA,
            pltpu.SemaphoreType.DMA,
            pltpu.SemaphoreType.DMA,
        ],
    )
    def unpool(data_hbm, idx_hbm, out_hbm,
               data0, data1, idx0, idx1, out0, out1,
               sd0, sd1, si0, si1, so0, so1):
        data_bufs, idx_bufs, out_bufs = (data0, data1), (idx0, idx1), (out0, out1)
        sd, si, so = (sd0, sd1), (si0, si1), (so0, so1)

        wid = lax.axis_index("s") * nc + lax.axis_index("c")
        g = wid // P
        t = wid % P
        ch0 = pl.multiple_of(g * CG, CG)
        nk = (nb_full - t + P - 1) // P

        iota = lax.broadcasted_iota(jnp.int32, (L,), 0)
        zed = jnp.zeros((L,), jnp.float32)

        def start_in(k, p):
            col0 = pl.multiple_of((k * P + t) * CB, 128)
            pltpu.async_copy(
                data_hbm.at[pl.ds(ch0, CG), pl.ds(col0, CB)], data_bufs[p], sd[p])
            pltpu.async_copy(
                idx_hbm.at[pl.ds(ch0, CG), pl.ds(col0, CB)], idx_bufs[p], si[p])

        def wait_in(p):
            pltpu.make_async_copy(
                data_hbm.at[pl.ds(0, CG), pl.ds(0, CB)], data_bufs[p], sd[p]).wait()
            pltpu.make_async_copy(
                idx_hbm.at[pl.ds(0, CG), pl.ds(0, CB)], idx_bufs[p], si[p]).wait()

        def start_out(k, p):
            col0 = pl.multiple_of((k * P + t) * OB, 128)
            pltpu.async_copy(
                out_bufs[p], out_hbm.at[pl.ds(ch0, CG), pl.ds(col0, OB)], so[p])

        def wait_out(p):
            pltpu.make_async_copy(
                out_bufs[p], out_hbm.at[pl.ds(0, CG), pl.ds(0, OB)], so[p]).wait()

        def compute(p, nvec):
            dv, iv, ov = data_bufs[p], idx_bufs[p], out_bufs[p]
            for c in range(CG):
                def zero_body(v, carry):
                    for u in range(E):
                        ov[c, pl.ds((v * E + u) * L, L)] = zed
                    return carry

                lax.fori_loop(0, nvec, zero_body, 0)

                cvec = jnp.full((L,), c, jnp.int32)

                def scat_body(v, carry):
                    d = dv[c, pl.ds(v * L, L)]
                    ix = iv[c, pl.ds(v * L, L)] & 7
                    pos = ix + (v * (L * E) + iota * E)
                    plsc.store_scatter(ov, [cvec, pos], d)
                    return carry

                lax.fori_loop(0, nvec, scat_body, 0)

        @pl.when(nk > 0)
        def _():
            start_in(0, 0)

        def body(i, carry):
            k0 = i * 2
            for p in range(2):
                k = k0 + p

                @pl.when(k < nk)
                def _():
                    wait_in(p)

                    @pl.when(k + 1 < nk)
                    def _():
                        start_in(k + 1, 1 - p)

                    compute(p, CB // L)

            return carry

        lax.fori_loop(0, (kmax + 1) // 2, body, 0)


        if rem:
            assert rem % L == 0
            rem_pad = ((rem + 127) // 128) * 128

            @pl.when(t == nb_full % P)
            def _():
                col0 = pl.multiple_of(nb_full * CB, 128)
                pltpu.sync_copy(
                    data_hbm.at[pl.ds(ch0, CG), pl.ds(col0, rem_pad)],
                    data0.at[:, pl.ds(0, rem_pad)])
                pltpu.sync_copy(
                    idx_hbm.at[pl.ds(ch0, CG), pl.ds(col0, rem_pad)],
                    idx0.at[:, pl.ds(0, rem_pad)])
                compute(0, rem_pad // L)
                pltpu.sync_copy(
                    out0.at[:, pl.ds(0, rem * E)],
                    out_hbm.at[pl.ds(ch0, CG), pl.ds(col0 * E, rem * E)])

    return unpool


def kernel(data, indices, depth):
    num, channel = data.shape
    unpool = _make_sc_unpool(num, channel)
    out_t = unpool(data.T, indices.astype(jnp.int32).T)
    return out_t.T


# speedup vs baseline: 1.5431x; 1.5431x over previous
import functools

import jax
import jax.numpy as jnp
from jax import lax
from jax.experimental import pallas as pl
from jax.experimental.pallas import tpu as pltpu
from jax.experimental.pallas import tpu_sc as plsc

E = 8
L = 16
CG = 8
CB = 512
OB = CB * E


def _make_sc_unpool(num: int, channel: int):
    info = plsc.get_sparse_core_info()
    nc, ns = info.num_cores, info.num_subcores
    nw = nc * ns
    ngroups = channel // CG
    P = nw // ngroups
    nb_full = num // CB
    rem = num - nb_full * CB
    kmax = (nb_full + P - 1) // P

    mesh = plsc.VectorSubcoreMesh(core_axis_name="c", subcore_axis_name="s")

    @functools.partial(
        pl.kernel,
        mesh=mesh,
        compiler_params=pltpu.CompilerParams(needs_layout_passes=False),
        out_type=jax.ShapeDtypeStruct((channel, num * E), jnp.float32),
        scratch_types=[
            pltpu.VMEM((CG, CB), jnp.float32),
            pltpu.VMEM((CG, CB), jnp.float32),
            pltpu.VMEM((CG, CB), jnp.int32),
            pltpu.VMEM((CG, CB), jnp.int32),
            pltpu.VMEM((CG, OB), jnp.float32),
            pltpu.VMEM((CG, OB), jnp.float32),
            pltpu.SemaphoreType.DMA,
            pltpu.SemaphoreType.DMA,
            pltpu.SemaphoreType.DMA,
            pltpu.SemaphoreType.DMA,
            pltpu.SemaphoreType.DMA,
            pltpu.SemaphoreType.DMA,
        ],
    )
    def unpool(data_hbm, idx_hbm, out_hbm,
               data0, data1, idx0, idx1, out0, out1,
               sd0, sd1, si0, si1, so0, so1):
        data_bufs, idx_bufs, out_bufs = (data0, data1), (idx0, idx1), (out0, out1)
        sd, si, so = (sd0, sd1), (si0, si1), (so0, so1)

        wid = lax.axis_index("s") * nc + lax.axis_index("c")
        g = wid // P
        t = wid % P
        ch0 = pl.multiple_of(g * CG, CG)
        nk = (nb_full - t + P - 1) // P

        iota = lax.broadcasted_iota(jnp.int32, (L,), 0)
        zed = jnp.zeros((L,), jnp.float32)

        def start_in(k, p):
            col0 = pl.multiple_of((k * P + t) * CB, 128)
            pltpu.async_copy(
                data_hbm.at[pl.ds(ch0, CG), pl.ds(col0, CB)], data_bufs[p], sd[p])
            pltpu.async_copy(
                idx_hbm.at[pl.ds(ch0, CG), pl.ds(col0, CB)], idx_bufs[p], si[p])

        def wait_in(p):
            pltpu.make_async_copy(
                data_hbm.at[pl.ds(0, CG), pl.ds(0, CB)], data_bufs[p], sd[p]).wait()
            pltpu.make_async_copy(
                idx_hbm.at[pl.ds(0, CG), pl.ds(0, CB)], idx_bufs[p], si[p]).wait()

        def start_out(k, p):
            col0 = pl.multiple_of((k * P + t) * OB, 128)
            pltpu.async_copy(
                out_bufs[p], out_hbm.at[pl.ds(ch0, CG), pl.ds(col0, OB)], so[p])

        def wait_out(p):
            pltpu.make_async_copy(
                out_bufs[p], out_hbm.at[pl.ds(0, CG), pl.ds(0, OB)], so[p]).wait()

        cvecs = [jnp.full((L,), c, jnp.int32) for c in range(CG)]
        iota_e = iota * E

        def compute(p, nvec):
            dv, iv, ov = data_bufs[p], idx_bufs[p], out_bufs[p]

            @plsc.parallel_loop(0, nvec)
            def _zero(v):
                for c in range(CG):
                    for u in range(E):
                        ov[c, pl.ds((v * E + u) * L, L)] = zed

            @plsc.parallel_loop(0, nvec)
            def _scat(v):
                base = iota_e + v * (L * E)
                for c in range(CG):
                    d = dv[c, pl.ds(v * L, L)]
                    ix = iv[c, pl.ds(v * L, L)] & 7
                    plsc.store_scatter(ov, [cvecs[c], ix + base], d)

        @pl.when(nk > 0)
        def _():
            start_in(0, 0)

        def body(i, carry):
            k0 = i * 2
            for p in range(2):
                k = k0 + p

                @pl.when(k < nk)
                def _():
                    wait_in(p)

                    @pl.when(k + 1 < nk)
                    def _():
                        start_in(k + 1, 1 - p)

                    @pl.when(k >= 2)
                    def _():
                        wait_out(p)

                    compute(p, CB // L)
                    start_out(k, p)

            return carry

        lax.fori_loop(0, (kmax + 1) // 2, body, 0)

        @pl.when(nk >= 1)
        def _():
            wait_out(0)

        @pl.when(nk >= 2)
        def _():
            wait_out(1)

        if rem:
            assert rem % L == 0
            rem_pad = ((rem + 127) // 128) * 128

            @pl.when(t == nb_full % P)
            def _():
                col0 = pl.multiple_of(nb_full * CB, 128)
                pltpu.sync_copy(
                    data_hbm.at[pl.ds(ch0, CG), pl.ds(col0, rem_pad)],
                    data0.at[:, pl.ds(0, rem_pad)])
                pltpu.sync_copy(
                    idx_hbm.at[pl.ds(ch0, CG), pl.ds(col0, rem_pad)],
                    idx0.at[:, pl.ds(0, rem_pad)])
                compute(0, rem_pad // L)
                pltpu.sync_copy(
                    out0.at[:, pl.ds(0, rem * E)],
                    out_hbm.at[pl.ds(ch0, CG), pl.ds(col0 * E, rem * E)])

    return unpool


def kernel(data, indices, depth):
    num, channel = data.shape
    unpool = _make_sc_unpool(num, channel)
    out_t = unpool(data.T, indices.astype(jnp.int32).T)
    return out_t.T


# speedup vs baseline: 1.5441x; 1.0007x over previous
import functools

import jax
import jax.numpy as jnp
from jax import lax
from jax.experimental import pallas as pl
from jax.experimental.pallas import tpu as pltpu
from jax.experimental.pallas import tpu_sc as plsc

E = 8
L = 16
CG = 8
CB = 768
OB = CB * E


def _make_sc_unpool(num: int, channel: int):
    info = plsc.get_sparse_core_info()
    nc, ns = info.num_cores, info.num_subcores
    nw = nc * ns
    ngroups = channel // CG
    P = nw // ngroups
    nb_full = num // CB
    rem = num - nb_full * CB
    kmax = (nb_full + P - 1) // P

    mesh = plsc.VectorSubcoreMesh(core_axis_name="c", subcore_axis_name="s")

    @functools.partial(
        pl.kernel,
        mesh=mesh,
        compiler_params=pltpu.CompilerParams(needs_layout_passes=False),
        out_type=jax.ShapeDtypeStruct((channel, num * E), jnp.float32),
        scratch_types=[
            pltpu.VMEM((CG, CB), jnp.float32),
            pltpu.VMEM((CG, CB), jnp.float32),
            pltpu.VMEM((CG, CB), jnp.int32),
            pltpu.VMEM((CG, CB), jnp.int32),
            pltpu.VMEM((CG, OB), jnp.float32),
            pltpu.VMEM((CG, OB), jnp.float32),
            pltpu.SemaphoreType.DMA,
            pltpu.SemaphoreType.DMA,
            pltpu.SemaphoreType.DMA,
            pltpu.SemaphoreType.DMA,
            pltpu.SemaphoreType.DMA,
            pltpu.SemaphoreType.DMA,
        ],
    )
    def unpool(data_hbm, idx_hbm, out_hbm,
               data0, data1, idx0, idx1, out0, out1,
               sd0, sd1, si0, si1, so0, so1):
        data_bufs, idx_bufs, out_bufs = (data0, data1), (idx0, idx1), (out0, out1)
        sd, si, so = (sd0, sd1), (si0, si1), (so0, so1)

        wid = lax.axis_index("s") * nc + lax.axis_index("c")
        g = wid // P
        t = wid % P
        ch0 = pl.multiple_of(g * CG, CG)
        nk = (nb_full - t + P - 1) // P

        iota = lax.broadcasted_iota(jnp.int32, (L,), 0)
        zed = jnp.zeros((L,), jnp.float32)

        def start_in(k, p):
            col0 = pl.multiple_of((k * P + t) * CB, 128)
            pltpu.async_copy(
                data_hbm.at[pl.ds(ch0, CG), pl.ds(col0, CB)], data_bufs[p], sd[p])
            pltpu.async_copy(
                idx_hbm.at[pl.ds(ch0, CG), pl.ds(col0, CB)], idx_bufs[p], si[p])

        def wait_in(p):
            pltpu.make_async_copy(
                data_hbm.at[pl.ds(0, CG), pl.ds(0, CB)], data_bufs[p], sd[p]).wait()
            pltpu.make_async_copy(
                idx_hbm.at[pl.ds(0, CG), pl.ds(0, CB)], idx_bufs[p], si[p]).wait()

        def start_out(k, p):
            col0 = pl.multiple_of((k * P + t) * OB, 128)
            pltpu.async_copy(
                out_bufs[p], out_hbm.at[pl.ds(ch0, CG), pl.ds(col0, OB)], so[p])

        def wait_out(p):
            pltpu.make_async_copy(
                out_bufs[p], out_hbm.at[pl.ds(0, CG), pl.ds(0, OB)], so[p]).wait()

        cvecs = [jnp.full((L,), c, jnp.int32) for c in range(CG)]
        iota_e = iota * E

        def compute(p, nvec):
            dv, iv, ov = data_bufs[p], idx_bufs[p], out_bufs[p]

            @plsc.parallel_loop(0, nvec)
            def _zero(v):
                for c in range(CG):
                    for u in range(E):
                        ov[c, pl.ds((v * E + u) * L, L)] = zed

            @plsc.parallel_loop(0, nvec)
            def _scat(v):
                base = iota_e + v * (L * E)
                for c in range(CG):
                    d = dv[c, pl.ds(v * L, L)]
                    ix = iv[c, pl.ds(v * L, L)] & 7
                    plsc.store_scatter(ov, [cvecs[c], ix + base], d)

        @pl.when(nk > 0)
        def _():
            start_in(0, 0)

        def body(i, carry):
            k0 = i * 2
            for p in range(2):
                k = k0 + p

                @pl.when(k < nk)
                def _():
                    wait_in(p)

                    @pl.when(k + 1 < nk)
                    def _():
                        start_in(k + 1, 1 - p)

                    @pl.when(k >= 2)
                    def _():
                        wait_out(p)

                    compute(p, CB // L)
                    start_out(k, p)

            return carry

        lax.fori_loop(0, (kmax + 1) // 2, body, 0)

        @pl.when(nk >= 1)
        def _():
            wait_out(0)

        @pl.when(nk >= 2)
        def _():
            wait_out(1)

        if rem:
            assert rem % L == 0
            rem_pad = ((rem + 127) // 128) * 128

            @pl.when(t == nb_full % P)
            def _():
                col0 = pl.multiple_of(nb_full * CB, 128)
                pltpu.sync_copy(
                    data_hbm.at[pl.ds(ch0, CG), pl.ds(col0, rem_pad)],
                    data0.at[:, pl.ds(0, rem_pad)])
                pltpu.sync_copy(
                    idx_hbm.at[pl.ds(ch0, CG), pl.ds(col0, rem_pad)],
                    idx0.at[:, pl.ds(0, rem_pad)])
                compute(0, rem_pad // L)
                pltpu.sync_copy(
                    out0.at[:, pl.ds(0, rem * E)],
                    out_hbm.at[pl.ds(ch0, CG), pl.ds(col0 * E, rem * E)])

    return unpool


def kernel(data, indices, depth):
    num, channel = data.shape
    unpool = _make_sc_unpool(num, channel)
    out_t = unpool(data.T, indices.astype(jnp.int32).T)
    return out_t.T
